# baseline (speedup 1.0000x reference)
import functools

import jax
import jax.numpy as jnp
from jax import lax
from jax.experimental import pallas as pl
from jax.experimental.pallas import tpu as pltpu
from jax.experimental.pallas import tpu_sc as plsc

NC = 2
NS = 16
NW = NC * NS
L = 16
K = 128
G = K // L
R = 8


def _splat(v):
    return jnp.full((L,), v, jnp.int32)


def _make_sc_kernel(n_pad, t_dim, h_dim, w_dim, n_chunks):
    hw = h_dim * w_dim
    bw = n_pad // NW
    mesh = plsc.VectorSubcoreMesh(core_axis_name="c", subcore_axis_name="s",
                                  num_cores=NC, num_subcores=NS)

    @functools.partial(
        pl.kernel,
        out_type=jax.ShapeDtypeStruct((n_pad, 3), jnp.float32),
        mesh=mesh,
        scratch_types=[
            [pltpu.VMEM((K,), jnp.float32) for _ in range(3)],
            [pltpu.VMEM((K,), jnp.int32) for _ in range(4)],
            [pltpu.VMEM((K, R), jnp.float32) for _ in range(4)],
            [pltpu.VMEM((K,), jnp.float32) for _ in range(3)],
            pltpu.VMEM((K, 3), jnp.float32),
            pltpu.SemaphoreType.DMA,
        ],
        compiler_params=pltpu.CompilerParams(
            needs_layout_passes=False, use_tc_tiling_on_sc=False),
    )
    def video_kernel(xs_hbm, ys_hbm, ts_hbm, tab_hbm, out_hbm,
                     pts, idx, gbuf, wgt, obuf, sem):
        wid = lax.axis_index("s") * NC + lax.axis_index("c")
        base = wid * bw

        def chunk_body(ci, carry):
            start = base + ci * K
            pltpu.sync_copy(xs_hbm.at[pl.ds(start, K)], pts[0])
            pltpu.sync_copy(ys_hbm.at[pl.ds(start, K)], pts[1])
            pltpu.sync_copy(ts_hbm.at[pl.ds(start, K)], pts[2])

            for g in range(G):
                sl = pl.ds(g * L, L)
                x = pts[0][sl] * w_dim
                y = pts[1][sl] * h_dim
                t = pts[2][sl] * t_dim
                xi = x.astype(jnp.int32)
                yi = y.astype(jnp.int32)
                ti = t.astype(jnp.int32)
                wgt[0][sl] = x - xi.astype(jnp.float32)
                wgt[1][sl] = y - yi.astype(jnp.float32)
                wgt[2][sl] = t - ti.astype(jnp.float32)
                x0 = jnp.minimum(xi, w_dim - 1)
                y0 = jnp.minimum(yi, h_dim - 1)
                t0 = jnp.minimum(ti, t_dim - 1)
                dy = (jnp.minimum(y0 + 1, h_dim - 1) - y0) * w_dim
                dt = (jnp.minimum(t0 + 1, t_dim - 1) - t0) * hw
                b0 = (t0 * h_dim + y0) * w_dim + x0
                idx[0][sl] = b0
                idx[1][sl] = b0 + dy
                idx[2][sl] = b0 + dt
                idx[3][sl] = b0 + dy + dt

            cps = [
                pltpu.async_copy(tab_hbm.at[idx[j]], gbuf[j], sem)
                for j in range(4)
            ]
            for cp in cps:
                cp.wait()

            for g in range(G):
                sl = pl.ds(g * L, L)
                rows = lax.iota(jnp.int32, L) + g * L
                fx = wgt[0][sl]
                fy = wgt[1][sl]
                ft = wgt[2][sl]
                gx = 1.0 - fx
                gy = 1.0 - fy
                gt = 1.0 - ft
                for ch in range(3):
                    c0 = _splat(ch)
                    c1 = _splat(ch + 4)

                    def pair(j):
                        v0 = plsc.load_gather(gbuf[j], [rows, c0])
                        v1 = plsc.load_gather(gbuf[j], [rows, c1])
                        return gx * v0 + fx * v1

                    acc = ((pair(0) * gy + pair(1) * fy) * gt +
                           (pair(2) * gy + pair(3) * fy) * ft)
                    plsc.store_scatter(obuf, [rows, c0], acc)

            pltpu.sync_copy(obuf, out_hbm.at[pl.ds(start, K)])
            return carry

        lax.fori_loop(0, n_chunks, chunk_body, 0)

    return video_kernel


def kernel(xyt, data):
    t_dim, h_dim, w_dim, c_dim = data.shape
    n = xyt.shape[0]
    n_chunks = -(-n // (NW * K))
    n_pad = NW * K * n_chunks
    xyt_p = jnp.pad(xyt, ((0, n_pad - n), (0, 0)))
    xs = xyt_p[:, 0]
    ys = xyt_p[:, 1]
    ts = xyt_p[:, 2]
    nxt = jnp.concatenate([data[:, :, 1:], data[:, :, -1:]], axis=2)
    zc = jnp.zeros(data.shape[:3] + (1,), jnp.float32)
    pair_tab = jnp.concatenate([data, zc, nxt, zc], axis=-1)
    pair_tab = pair_tab.reshape(t_dim * h_dim * w_dim, R)
    f = _make_sc_kernel(n_pad, t_dim, h_dim, w_dim, n_chunks)
    out = f(xs, ys, ts, pair_tab)
    return out[:n]


# speedup vs baseline: 4.9796x; 4.9796x over previous
import functools

import jax
import jax.numpy as jnp
from jax import lax
from jax.experimental import pallas as pl
from jax.experimental.pallas import tpu as pltpu
from jax.experimental.pallas import tpu_sc as plsc

NC = 2
NS = 16
NW = NC * NS
L = 16
K = 128
G = K // L
R = 8
TY = 8
TX = 128


def _make_sc_kernel(n_pad, t_dim, h_dim, w_dim, c_dim, n_chunks):
    hw = h_dim * w_dim
    ntc = w_dim // TX
    tr_stride = ntc * TY * TX
    bw = n_pad // NW
    mesh = plsc.VectorSubcoreMesh(core_axis_name="c", subcore_axis_name="s",
                                  num_cores=NC, num_subcores=NS)

    @functools.partial(
        pl.kernel,
        out_type=[jax.ShapeDtypeStruct((n_pad,), jnp.float32)
                  for _ in range(3)],
        mesh=mesh,
        scratch_types=[
            [pltpu.VMEM((K,), jnp.float32) for _ in range(3)],
            [pltpu.VMEM((K,), jnp.int32) for _ in range(24)],
            [pltpu.VMEM((K,), jnp.int32) for _ in range(2)],
            [pltpu.VMEM((K, R), jnp.float32) for _ in range(24)],
            [pltpu.VMEM((K,), jnp.float32) for _ in range(3)],
            [pltpu.VMEM((K,), jnp.float32) for _ in range(3)],
            pltpu.SemaphoreType.DMA,
        ],
        compiler_params=pltpu.CompilerParams(
            needs_layout_passes=False, use_tc_tiling_on_sc=False),
    )
    def video_kernel(xs_hbm, ys_hbm, ts_hbm, tab_hbm,
                     o0_hbm, o1_hbm, o2_hbm,
                     pts, idx, off, gbuf, wgt, obuf, sem):
        wid = lax.axis_index("s") * NC + lax.axis_index("c")
        base = wid * bw

        def chunk_body(ci, carry):
            start = base + ci * K
            pltpu.sync_copy(xs_hbm.at[pl.ds(start, K)], pts[0])
            pltpu.sync_copy(ys_hbm.at[pl.ds(start, K)], pts[1])
            pltpu.sync_copy(ts_hbm.at[pl.ds(start, K)], pts[2])

            for g in range(G):
                sl = pl.ds(g * L, L)
                x = pts[0][sl] * w_dim
                y = pts[1][sl] * h_dim
                t = pts[2][sl] * t_dim
                xi = x.astype(jnp.int32)
                yi = y.astype(jnp.int32)
                ti = t.astype(jnp.int32)
                wgt[0][sl] = x - xi.astype(jnp.float32)
                wgt[1][sl] = y - yi.astype(jnp.float32)
                wgt[2][sl] = t - ti.astype(jnp.float32)
                x0 = jnp.minimum(xi, w_dim - 1)
                y0 = jnp.minimum(yi, h_dim - 1)
                t0 = jnp.minimum(ti, t_dim - 1)
                x1 = jnp.minimum(x0 + 1, w_dim - 1)
                y1 = jnp.minimum(y0 + 1, h_dim - 1)
                t1 = jnp.minimum(t0 + 1, t_dim - 1)
                off[0][sl] = jnp.bitwise_and(x0, 7)
                off[1][sl] = jnp.bitwise_and(x1, 7)
                ya = (lax.shift_right_logical(y0, 3) * tr_stride +
                      jnp.bitwise_and(y0, 7) * TX)
                yb = (lax.shift_right_logical(y1, 3) * tr_stride +
                      jnp.bitwise_and(y1, 7) * TX)
                xa = (lax.shift_right_logical(x0, 7) * (TY * TX) +
                      jnp.bitwise_and(x0, 127))
                xb = (lax.shift_right_logical(x1, 7) * (TY * TX) +
                      jnp.bitwise_and(x1, 127))
                for c in range(c_dim):
                    pa = (t0 * c_dim + c) * hw
                    pb = (t1 * c_dim + c) * hw
                    for tt, pp in enumerate((pa, pb)):
                        for yy, aa in enumerate((ya, yb)):
                            j = ((tt * 2 + yy) * 3 + c) * 2
                            w0 = pp + aa + xa
                            w1 = pp + aa + xb
                            idx[j][sl] = lax.shift_right_logical(w0, 3)
                            idx[j + 1][sl] = lax.shift_right_logical(w1, 3)

            cps = [pltpu.async_copy(tab_hbm.at[idx[j]], gbuf[j], sem)
                   for j in range(24)]
            for cp in cps:
                cp.wait()

            for g in range(G):
                sl = pl.ds(g * L, L)
                rows = lax.iota(jnp.int32, L) + g * L
                fx = wgt[0][sl]
                fy = wgt[1][sl]
                ft = wgt[2][sl]
                gx = 1.0 - fx
                gy = 1.0 - fy
                gt = 1.0 - ft
                o0v = off[0][sl]
                o1v = off[1][sl]
                wy00 = gy * gt
                wy01 = fy * gt
                wy10 = gy * ft
                wy11 = fy * ft
                for c in range(c_dim):
                    def pix(tt, yy):
                        j = ((tt * 2 + yy) * 3 + c) * 2
                        vA = plsc.load_gather(gbuf[j], [rows, o0v])
                        vB = plsc.load_gather(gbuf[j + 1], [rows, o1v])
                        return gx * vA + fx * vB

                    acc = (pix(0, 0) * wy00 + pix(0, 1) * wy01 +
                           pix(1, 0) * wy10 + pix(1, 1) * wy11)
                    obuf[c][sl] = acc

            pltpu.sync_copy(obuf[0], o0_hbm.at[pl.ds(start, K)])
            pltpu.sync_copy(obuf[1], o1_hbm.at[pl.ds(start, K)])
            pltpu.sync_copy(obuf[2], o2_hbm.at[pl.ds(start, K)])
            return carry

        lax.fori_loop(0, n_chunks, chunk_body, 0)

    return video_kernel


def kernel(xyt, data):
    t_dim, h_dim, w_dim, c_dim = data.shape
    n = xyt.shape[0]
    n_chunks = -(-n // (NW * K))
    n_pad = NW * K * n_chunks
    xyt_p = jnp.pad(xyt, ((0, n_pad - n), (0, 0)))
    xs = xyt_p[:, 0]
    ys = xyt_p[:, 1]
    ts = xyt_p[:, 2]
    z = data.reshape(t_dim, h_dim // TY, TY, w_dim // TX, TX, c_dim)
    z = z.transpose(0, 5, 1, 3, 2, 4)
    tab = z.reshape(t_dim * c_dim * h_dim * w_dim // R, R)
    f = _make_sc_kernel(n_pad, t_dim, h_dim, w_dim, c_dim, n_chunks)
    o0, o1, o2 = f(xs, ys, ts, tab)
    out = jnp.stack([o0[:n], o1[:n], o2[:n]], axis=1)
    return out


# speedup vs baseline: 7.4256x; 1.4912x over previous
import functools

import jax
import jax.numpy as jnp
from jax import lax
from jax.experimental import pallas as pl
from jax.experimental.pallas import tpu as pltpu
from jax.experimental.pallas import tpu_sc as plsc

NC = 2
NS = 16
NW = NC * NS
L = 16
K = 128
G = K // L
R = 8
TY = 8
TX = 128


def _make_sc_kernel(n_pad, t_dim, h_dim, w_dim, c_dim, n_chunks):
    hw = h_dim * w_dim
    ntc = w_dim // TX
    tr_stride = ntc * TY * TX
    bw = n_pad // NW
    nhalf = n_chunks // 2
    mesh = plsc.VectorSubcoreMesh(core_axis_name="c", subcore_axis_name="s",
                                  num_cores=NC, num_subcores=NS)

    def _set():
        return [
            [pltpu.VMEM((K,), jnp.float32) for _ in range(3)],
            [pltpu.VMEM((K,), jnp.int32) for _ in range(24)],
            [pltpu.VMEM((K,), jnp.int32) for _ in range(2)],
            [pltpu.VMEM((K, R), jnp.float32) for _ in range(24)],
            [pltpu.VMEM((K,), jnp.float32) for _ in range(3)],
            [pltpu.VMEM((K,), jnp.float32) for _ in range(3)],
            pltpu.SemaphoreType.DMA,
        ]

    @functools.partial(
        pl.kernel,
        out_type=[jax.ShapeDtypeStruct((n_pad,), jnp.float32)
                  for _ in range(3)],
        mesh=mesh,
        scratch_types=[_set(), _set()],
        compiler_params=pltpu.CompilerParams(
            needs_layout_passes=False, use_tc_tiling_on_sc=False),
    )
    def video_kernel(xs_hbm, ys_hbm, ts_hbm, tab_hbm,
                     o0_hbm, o1_hbm, o2_hbm, set0, set1):
        wid = lax.axis_index("s") * NC + lax.axis_index("c")
        base = wid * bw
        sets = (set0, set1)

        def stage_fire(ci, s):
            pts, idx, off, gbuf, wgt, obuf, sem = sets[s]
            start = base + ci * K
            pltpu.sync_copy(xs_hbm.at[pl.ds(start, K)], pts[0])
            pltpu.sync_copy(ys_hbm.at[pl.ds(start, K)], pts[1])
            pltpu.sync_copy(ts_hbm.at[pl.ds(start, K)], pts[2])
            for g in range(G):
                sl = pl.ds(g * L, L)
                x = pts[0][sl] * w_dim
                y = pts[1][sl] * h_dim
                t = pts[2][sl] * t_dim
                xi = x.astype(jnp.int32)
                yi = y.astype(jnp.int32)
                ti = t.astype(jnp.int32)
                wgt[0][sl] = x - xi.astype(jnp.float32)
                wgt[1][sl] = y - yi.astype(jnp.float32)
                wgt[2][sl] = t - ti.astype(jnp.float32)
                x0 = jnp.minimum(xi, w_dim - 1)
                y0 = jnp.minimum(yi, h_dim - 1)
                t0 = jnp.minimum(ti, t_dim - 1)
                x1 = jnp.minimum(x0 + 1, w_dim - 1)
                y1 = jnp.minimum(y0 + 1, h_dim - 1)
                t1 = jnp.minimum(t0 + 1, t_dim - 1)
                off[0][sl] = jnp.bitwise_and(x0, 7)
                off[1][sl] = jnp.bitwise_and(x1, 7)
                ya = (lax.shift_right_logical(y0, 3) * tr_stride +
                      jnp.bitwise_and(y0, 7) * TX)
                yb = (lax.shift_right_logical(y1, 3) * tr_stride +
                      jnp.bitwise_and(y1, 7) * TX)
                xa = (lax.shift_right_logical(x0, 7) * (TY * TX) +
                      jnp.bitwise_and(x0, 127))
                xb = (lax.shift_right_logical(x1, 7) * (TY * TX) +
                      jnp.bitwise_and(x1, 127))
                for c in range(c_dim):
                    pa = (t0 * c_dim + c) * hw
                    pb = (t1 * c_dim + c) * hw
                    for tt, pp in enumerate((pa, pb)):
                        for yy, aa in enumerate((ya, yb)):
                            j = ((tt * 2 + yy) * 3 + c) * 2
                            w0 = pp + aa + xa
                            w1 = pp + aa + xb
                            idx[j][sl] = lax.shift_right_logical(w0, 3)
                            idx[j + 1][sl] = lax.shift_right_logical(w1, 3)
            for j in range(24):
                pltpu.async_copy(tab_hbm.at[idx[j]], gbuf[j], sem)

        def drain_blend(ci, s):
            pts, idx, off, gbuf, wgt, obuf, sem = sets[s]
            start = base + ci * K
            for j in range(24):
                pltpu.make_async_copy(tab_hbm.at[idx[j]], gbuf[j], sem).wait()
            for g in range(G):
                sl = pl.ds(g * L, L)
                rows = lax.iota(jnp.int32, L) + g * L
                fx = wgt[0][sl]
                fy = wgt[1][sl]
                ft = wgt[2][sl]
                gx = 1.0 - fx
                gy = 1.0 - fy
                gt = 1.0 - ft
                o0v = off[0][sl]
                o1v = off[1][sl]
                wy00 = gy * gt
                wy01 = fy * gt
                wy10 = gy * ft
                wy11 = fy * ft
                for c in range(c_dim):
                    def pix(tt, yy):
                        j = ((tt * 2 + yy) * 3 + c) * 2
                        vA = plsc.load_gather(gbuf[j], [rows, o0v])
                        vB = plsc.load_gather(gbuf[j + 1], [rows, o1v])
                        return gx * vA + fx * vB

                    acc = (pix(0, 0) * wy00 + pix(0, 1) * wy01 +
                           pix(1, 0) * wy10 + pix(1, 1) * wy11)
                    obuf[c][sl] = acc
            pltpu.sync_copy(obuf[0], o0_hbm.at[pl.ds(start, K)])
            pltpu.sync_copy(obuf[1], o1_hbm.at[pl.ds(start, K)])
            pltpu.sync_copy(obuf[2], o2_hbm.at[pl.ds(start, K)])

        stage_fire(jnp.int32(0), 0)

        def body(i, carry):
            stage_fire(2 * i + 1, 1)
            drain_blend(2 * i, 0)

            @pl.when(i < nhalf - 1)
            def _():
                stage_fire(2 * i + 2, 0)

            drain_blend(2 * i + 1, 1)
            return carry

        lax.fori_loop(0, nhalf, body, 0)

    return video_kernel


def kernel(xyt, data):
    t_dim, h_dim, w_dim, c_dim = data.shape
    n = xyt.shape[0]
    n_chunks = -(-n // (NW * K))
    n_chunks += n_chunks % 2
    n_pad = NW * K * n_chunks
    xyt_p = jnp.pad(xyt, ((0, n_pad - n), (0, 0)))
    xs = xyt_p[:, 0]
    ys = xyt_p[:, 1]
    ts = xyt_p[:, 2]
    z = data.reshape(t_dim, h_dim // TY, TY, w_dim // TX, TX, c_dim)
    z = z.transpose(0, 5, 1, 3, 2, 4)
    tab = z.reshape(t_dim * c_dim * h_dim * w_dim // R, R)
    f = _make_sc_kernel(n_pad, t_dim, h_dim, w_dim, c_dim, n_chunks)
    o0, o1, o2 = f(xs, ys, ts, tab)
    out = jnp.stack([o0[:n], o1[:n], o2[:n]], axis=1)
    return out
